# baseline (speedup 1.0000x reference)
import functools

import jax
import jax.numpy as jnp
from jax import lax
from jax.experimental import pallas as pl
from jax.experimental.pallas import tpu as pltpu
from jax.experimental.pallas import tpu_sc as plsc

ROWS = 128
COLS = 8192
TOPK = 256
LANES = 16
NCORES = 2
NSUB = 16
NWORKERS = NCORES * NSUB
RPW = ROWS // NWORKERS
VECS = COLS // LANES
NBUCKETS = 256
NCHUNKS = NBUCKETS // LANES
ABSMASK = jnp.int32(0x7FFFFFFF)
CROW = COLS + LANES


def _pick(counts, k):
    cs = jnp.cumsum(counts)
    total = jnp.max(cs)
    ea = total - cs
    idx = jnp.sum((ea >= k).astype(jnp.int32))
    above = jnp.max(jnp.where(ea < k, ea, jnp.int32(0)))
    return idx, above


def _tec_body(x_hbm, out_hbm, rows_v, hist_v, h16_v, cand_v):
    wid = lax.axis_index("s") * NCORES + lax.axis_index("c")
    base = wid * RPW
    pltpu.sync_copy(x_hbm.at[pl.ds(base, RPW)], rows_v)

    iota = lax.iota(jnp.int32, LANES)
    ones = jnp.ones((LANES,), jnp.int32)
    zeros16 = jnp.zeros((LANES,), jnp.int32)

    def _zero_hists():
        for r in range(RPW):
            for c in range(NCHUNKS):
                hist_v[pl.ds(r * NBUCKETS + c * LANES, LANES)] = zeros16
        for c in range(RPW):
            h16_v[pl.ds(c * LANES, LANES)] = zeros16

    def _select(r, k):
        chunk_sums = zeros16
        for c in range(NCHUNKS):
            s = jnp.sum(hist_v[pl.ds(r * NBUCKETS + c * LANES, LANES)])
            chunk_sums = chunk_sums + jnp.where(iota == c, s, jnp.int32(0))
        c0, above_c = _pick(chunk_sums, k)
        k2 = k - above_c
        h = hist_v[pl.ds(r * NBUCKETS + c0 * LANES, LANES)]
        b0, above_in = _pick(h, k2)
        return c0 * LANES + b0, k2 - above_in

    _zero_hists()

    @plsc.parallel_loop(0, VECS, unroll=4)
    def _p1(j):
        sl = pl.ds(j * LANES, LANES)
        for r in range(RPW):
            ab = lax.bitcast_convert_type(rows_v[r, sl], jnp.int32) & ABSMASK
            d = lax.shift_right_logical(ab, 23)
            plsc.addupdate_scatter(hist_v, [d + jnp.int32(r * NBUCKETS)], ones)
            plsc.addupdate_scatter(
                h16_v,
                [lax.shift_right_logical(d, 4) + jnp.int32(r * LANES)], ones)

    b1 = []
    krem = []
    for r in range(RPW):
        b, kn = _select(r, jnp.int32(TOPK))
        b1.append(b)
        krem.append(kn)

    @plsc.parallel_loop(0, VECS, unroll=2, carry=(jnp.int32(0),) * RPW)
    def _comp(j, offs):
        sl = pl.ds(j * LANES, LANES)
        nxt = []
        for r in range(RPW):
            ab = lax.bitcast_convert_type(rows_v[r, sl], jnp.int32) & ABSMASK
            msk = lax.shift_right_logical(ab, 23) == b1[r]
            plsc.store_compressed(
                cand_v.at[pl.ds(offs[r] + jnp.int32(r * CROW), LANES)],
                ab, mask=msk)
            nxt.append(offs[r] + plsc.all_reduce_population_count(msk)[0])
        return tuple(nxt)

    ncand = _comp
    nvec = [lax.shift_right_logical(ncand[r] + (LANES - 1), 4)
            for r in range(RPW)]
    nvmax = jnp.maximum(jnp.maximum(nvec[0], nvec[1]),
                        jnp.maximum(nvec[2], nvec[3]))

    prefix = list(b1)
    for shift, width in ((15, 8), (7, 8), (0, 7)):
        _zero_hists()
        top = shift + width

        @plsc.parallel_loop(0, nvmax, unroll=1)
        def _cs(j, shift=shift, top=top, width=width, pfx=tuple(prefix)):
            sl = pl.ds(j * LANES, LANES)
            lane = j * LANES + iota
            for r in range(RPW):
                ab = cand_v[pl.ds(j * LANES + r * CROW, LANES)]
                msk = (lane < ncand[r]) & (
                    lax.shift_right_logical(ab, top) == pfx[r])
                d = lax.shift_right_logical(ab, shift) & jnp.int32(
                    (1 << width) - 1)
                plsc.addupdate_scatter(
                    hist_v, [d + jnp.int32(r * NBUCKETS)], ones, mask=msk)
                plsc.addupdate_scatter(
                    h16_v,
                    [lax.shift_right_logical(d, 4) + jnp.int32(r * LANES)],
                    ones, mask=msk)

        for r in range(RPW):
            b, kn = _select(r, krem[r])
            krem[r] = kn
            prefix[r] = lax.shift_left(prefix[r], width) | b

    @plsc.parallel_loop(0, VECS, unroll=4)
    def _mask(j, thr=tuple(prefix)):
        sl = pl.ds(j * LANES, LANES)
        for r in range(RPW):
            v = rows_v[r, sl]
            ab = lax.bitcast_convert_type(v, jnp.int32) & ABSMASK
            rows_v[r, sl] = jnp.where(ab >= thr[r], v, jnp.float32(0.0))

    pltpu.sync_copy(rows_v, out_hbm.at[pl.ds(base, RPW)])


_topk_call = functools.partial(
    pl.kernel,
    mesh=plsc.VectorSubcoreMesh(core_axis_name="c", subcore_axis_name="s"),
    out_type=jax.ShapeDtypeStruct((ROWS, COLS), jnp.float32),
    scratch_types=[
        pltpu.VMEM((RPW, COLS), jnp.float32),
        pltpu.VMEM((RPW * NBUCKETS,), jnp.int32),
        pltpu.VMEM((RPW * LANES,), jnp.int32),
        pltpu.VMEM((RPW * CROW,), jnp.int32),
    ],
    compiler_params=pltpu.CompilerParams(needs_layout_passes=False),
)(_tec_body)


@jax.jit
def kernel(inputs):
    return _topk_call(inputs)
